# Initial kernel scaffold; baseline (speedup 1.0000x reference)
import jax
import jax.numpy as jnp
from jax.experimental import pallas as pl


def kernel(x, shuffle_idx):
    raise NotImplementedError("write your pallas kernel here")


# speedup vs baseline: 2.3616x; 2.3616x over previous
import jax
import jax.numpy as jnp
from jax.experimental import pallas as pl
from jax.experimental.pallas import tpu as pltpu

_F = 2048
_G = 32
_D = 64
_PACK = 4
_S = _G // _PACK
_SW = _PACK * _D
_NS_ITERS = 18


def _stats_kernel(x_ref, sum_ref, gram_ref):
    k = pl.program_id(1)

    @pl.when(k == 0)
    def _init():
        sum_ref[...] = jnp.zeros_like(sum_ref)
        gram_ref[...] = jnp.zeros_like(gram_ref)

    xb = x_ref[...]
    sum_ref[...] += jnp.sum(xb, axis=0, keepdims=True)[None]
    for s in range(_S):
        slab = xb[:, s * _SW:(s + 1) * _SW]
        gram_ref[0, s] += jax.lax.dot_general(
            slab, slab, (((0,), (0,)), ((), ())),
            preferred_element_type=jnp.float32)


def _whiten_kernel(gram_ref, mu_ref, muT_ref, n_ref, w_ref):
    n = n_ref[0]
    mu = mu_ref[0]
    muT = muT_ref[0]
    ri = jax.lax.broadcasted_iota(jnp.int32, (_SW, _SW), 0)
    ci = jax.lax.broadcasted_iota(jnp.int32, (_SW, _SW), 1)
    mask = ((ri // _D) == (ci // _D)).astype(jnp.float32)
    eye = (ri == ci).astype(jnp.float32)

    cov = (gram_ref[0] - n * (muT * mu)) * mask * (1.0 / _G)
    rowsum = jnp.sum(jnp.abs(cov), axis=-1, keepdims=True)
    s = jnp.maximum(jnp.max(rowsum), 1e-30)
    a = cov * (1.0 / s)

    y = a
    z = eye
    for _ in range(_NS_ITERS):
        zy = jax.lax.dot_general(z, y, (((1,), (0,)), ((), ())),
                                 preferred_element_type=jnp.float32)
        t = 1.5 * eye - 0.5 * zy
        y = jax.lax.dot_general(y, t, (((1,), (0,)), ((), ())),
                                preferred_element_type=jnp.float32)
        z = jax.lax.dot_general(t, z, (((1,), (0,)), ((), ())),
                                preferred_element_type=jnp.float32)
    w_ref[0] = z * jax.lax.rsqrt(s)


def _apply_kernel(x_ref, w_ref, mu_ref, o_ref):
    xb = x_ref[...]
    for s in range(_S):
        xc = xb[:, s * _SW:(s + 1) * _SW] - mu_ref[0, s * _SW:(s + 1) * _SW]
        o_ref[:, s * _SW:(s + 1) * _SW] = jax.lax.dot_general(
            xc, w_ref[s], (((1,), (0,)), ((), ())),
            preferred_element_type=jnp.float32)


def kernel(x, shuffle_idx):
    n_rows, f = x.shape
    assert f == _F
    idx = shuffle_idx.astype(jnp.int32)
    inv = jnp.argsort(idx)
    xs = jnp.take(x, idx, axis=1)

    blk = 1024
    p_par = 8
    k_seq = n_rows // (blk * p_par)

    sums, gram = pl.pallas_call(
        _stats_kernel,
        grid=(p_par, k_seq),
        in_specs=[pl.BlockSpec((blk, _F), lambda p, k: (p * k_seq + k, 0))],
        out_specs=[
            pl.BlockSpec((1, 1, _F), lambda p, k: (p, 0, 0)),
            pl.BlockSpec((1, _S, _SW, _SW), lambda p, k: (p, 0, 0, 0)),
        ],
        out_shape=[
            jax.ShapeDtypeStruct((p_par, 1, _F), jnp.float32),
            jax.ShapeDtypeStruct((p_par, _S, _SW, _SW), jnp.float32),
        ],
        compiler_params=pltpu.CompilerParams(
            dimension_semantics=("parallel", "arbitrary")),
    )(xs)

    colsum = jnp.sum(sums, axis=0)
    gram = jnp.sum(gram, axis=0)
    mean = colsum / n_rows
    mu_s = mean.reshape(_S, 1, _SW)
    muT_s = mean.reshape(_S, _SW, 1)
    n_arr = jnp.full((1,), float(n_rows), jnp.float32)

    w = pl.pallas_call(
        _whiten_kernel,
        grid=(_S,),
        in_specs=[
            pl.BlockSpec((1, _SW, _SW), lambda s: (s, 0, 0)),
            pl.BlockSpec((1, 1, _SW), lambda s: (s, 0, 0)),
            pl.BlockSpec((1, _SW, 1), lambda s: (s, 0, 0)),
            pl.BlockSpec(memory_space=pltpu.SMEM),
        ],
        out_specs=pl.BlockSpec((1, _SW, _SW), lambda s: (s, 0, 0)),
        out_shape=jax.ShapeDtypeStruct((_S, _SW, _SW), jnp.float32),
        compiler_params=pltpu.CompilerParams(
            dimension_semantics=("parallel",)),
    )(gram, mu_s, muT_s, n_arr)

    y = pl.pallas_call(
        _apply_kernel,
        grid=(p_par, k_seq),
        in_specs=[
            pl.BlockSpec((blk, _F), lambda p, k: (p * k_seq + k, 0)),
            pl.BlockSpec((_S, _SW, _SW), lambda p, k: (0, 0, 0)),
            pl.BlockSpec((1, _F), lambda p, k: (0, 0)),
        ],
        out_specs=pl.BlockSpec((blk, _F), lambda p, k: (p * k_seq + k, 0)),
        out_shape=jax.ShapeDtypeStruct((n_rows, _F), jnp.float32),
        compiler_params=pltpu.CompilerParams(
            dimension_semantics=("parallel", "arbitrary")),
    )(xs, w, mean)

    return jnp.take(y, inv, axis=1)
